# Initial kernel scaffold; baseline (speedup 1.0000x reference)
import jax
import jax.numpy as jnp
from jax.experimental import pallas as pl


def kernel(feat, xyz, masks, key_mask, W, b):
    raise NotImplementedError("write your pallas kernel here")


# speedup vs baseline: 16.0160x; 16.0160x over previous
import functools

import jax
import jax.numpy as jnp
from jax import lax
from jax.experimental import pallas as pl
from jax.experimental.pallas import tpu as pltpu
from jax.experimental.pallas import tpu_sc as plsc

_B, _N, _IN, _OUT = 4, 8192, 64, 64
_M, _RADIUS, _NS = 2048, 0.2, 32
_R2 = _RADIUS * _RADIUS
_NW = 32
_GPW = (_B * _M) // _NW
_PPB = _M // (_NW // _B)


def _fps_body(xyzT_ref, kmf_ref, idx_ref, sx_ref, sy_ref, sz_ref, dist_ref):
    lanei = lax.broadcasted_iota(jnp.int32, (_B, _N), 1)
    lanef = lanei.astype(jnp.float32)
    off = kmf_ref[...] * ((lanef + 10.0) * 10.0)
    sx_ref[...] = xyzT_ref[0] + off
    sy_ref[...] = xyzT_ref[1] + off
    sz_ref[...] = xyzT_ref[2] + off
    dist_ref[...] = jnp.full((_B, _N), 1e10, jnp.float32)

    lanem = lax.broadcasted_iota(jnp.int32, (_B, _M), 1)

    def body(i, far):
        idx_ref[...] = jnp.where(lanem == i, far, idx_ref[...])
        onehot = lanei == far
        sx = sx_ref[...]
        sy = sy_ref[...]
        sz = sz_ref[...]
        cx = jnp.sum(jnp.where(onehot, sx, 0.0), axis=1, keepdims=True)
        cy = jnp.sum(jnp.where(onehot, sy, 0.0), axis=1, keepdims=True)
        cz = jnp.sum(jnp.where(onehot, sz, 0.0), axis=1, keepdims=True)
        dx = sx - cx
        dy = sy - cy
        dz = sz - cz
        d = (dx * dx + dy * dy) + dz * dz
        dist = jnp.minimum(dist_ref[...], d)
        dist_ref[...] = dist
        maxv = jnp.max(dist, axis=1, keepdims=True)
        return jnp.min(jnp.where(dist == maxv, lanei, _N), axis=1, keepdims=True)

    lax.fori_loop(0, _M, body, jnp.zeros((_B, 1), jnp.int32))


def _fps(xyzT, kmf):
    return pl.pallas_call(
        _fps_body,
        out_shape=jax.ShapeDtypeStruct((_B, _M), jnp.int32),
        scratch_shapes=[pltpu.VMEM((_B, _N), jnp.float32)] * 4,
    )(xyzT, kmf)


def _xf_body(feat_ref, xyz_ref, wft_ref, wxt_ref, bias_ref, g_ref):
    g = jnp.dot(feat_ref[0], wft_ref[...], preferred_element_type=jnp.float32)
    x3 = xyz_ref[0]
    g = g + x3[:, 0:1] * wxt_ref[0:1, :]
    g = g + x3[:, 1:2] * wxt_ref[1:2, :]
    g = g + x3[:, 2:3] * wxt_ref[2:3, :]
    g_ref[0] = g + bias_ref[...]


def _xform(feat, xyz, wft, wxt, bias):
    return pl.pallas_call(
        _xf_body,
        grid=(_B,),
        in_specs=[
            pl.BlockSpec((1, _N, _IN), lambda b: (b, 0, 0)),
            pl.BlockSpec((1, _N, 3), lambda b: (b, 0, 0)),
            pl.BlockSpec((_IN, _OUT), lambda b: (0, 0)),
            pl.BlockSpec((3, _OUT), lambda b: (0, 0)),
            pl.BlockSpec((1, _OUT), lambda b: (0, 0)),
        ],
        out_specs=pl.BlockSpec((1, _N, _OUT), lambda b: (b, 0, 0)),
        out_shape=jax.ShapeDtypeStruct((_B, _N, _OUT), jnp.float32),
    )(feat, xyz, wft, wxt, bias)


def _bf16_round(v):
    u = plsc.bitcast(v, jnp.uint32)
    u = u + jnp.uint32(0x7FFF) + ((u >> jnp.uint32(16)) & jnp.uint32(1))
    u = u & jnp.uint32(0xFFFF0000)
    return plsc.bitcast(u, jnp.float32)


def _sc_body(xyzT, gh, fih, m0h, m1h, kmh, wxth,
             outh, nxh, nyh, nzh, nm0h, nm1h, nkmh,
             xb, yb, zb, axb, ayb, azb, p2b, mb0, mb1, kmb, fpsb, wxb,
             nxb, nyb, nzb, nm0b, nm1b, nkmb,
             idxb, gidx, gbuf, outb, sem):
    cid = lax.axis_index("c")
    sid = lax.axis_index("s")
    w = sid * 2 + cid
    b = w // (_NW // _B)
    m0 = (w % (_NW // _B)) * _PPB

    pltpu.sync_copy(xyzT.at[0, b], xb)
    pltpu.sync_copy(xyzT.at[1, b], yb)
    pltpu.sync_copy(xyzT.at[2, b], zb)
    pltpu.sync_copy(m0h.at[b], mb0)
    pltpu.sync_copy(m1h.at[b], mb1)
    pltpu.sync_copy(kmh.at[b], kmb)
    pltpu.sync_copy(fih.at[b, pl.ds(m0, _PPB)], fpsb)
    pltpu.sync_copy(wxth, wxb)

    def prep(t, _):
        s = t * 16
        vx = xb[pl.ds(s, 16)]
        vy = yb[pl.ds(s, 16)]
        vz = zb[pl.ds(s, 16)]
        axb[pl.ds(s, 16)] = _bf16_round(vx)
        ayb[pl.ds(s, 16)] = _bf16_round(vy)
        azb[pl.ds(s, 16)] = _bf16_round(vz)
        p2b[pl.ds(s, 16)] = (vx * vx + vy * vy) + vz * vz
        return 0

    lax.fori_loop(0, _N // 16, prep, 0)

    def mg(t, _):
        s = t * 16
        iv = fpsb[pl.ds(s, 16)]
        nxb[pl.ds(s, 16)] = plsc.load_gather(xb, [iv])
        nyb[pl.ds(s, 16)] = plsc.load_gather(yb, [iv])
        nzb[pl.ds(s, 16)] = plsc.load_gather(zb, [iv])
        nm0b[pl.ds(s, 16)] = plsc.load_gather(mb0, [iv])
        nm1b[pl.ds(s, 16)] = plsc.load_gather(mb1, [iv])
        nkmb[pl.ds(s, 16)] = plsc.load_gather(kmb, [iv])
        return 0

    lax.fori_loop(0, _PPB // 16, mg, 0)

    lane16 = lax.iota(jnp.int32, 16)
    r2 = jnp.float32(_R2)
    inv_ns = jnp.float32(1.0 / _NS)

    def group(i, _):
        nvx = nxb[pl.ds(i, 16)]
        nvy = nyb[pl.ds(i, 16)]
        nvz = nzb[pl.ds(i, 16)]
        ccx = nvx[0]
        ccy = nvy[0]
        ccz = nvz[0]
        bcx = _bf16_round(nvx)[0]
        bcy = _bf16_round(nvy)[0]
        bcz = _bf16_round(nvz)[0]
        c2 = (ccx * ccx + ccy * ccy) + ccz * ccz

        def cond(carry):
            n, cnt = carry
            return jnp.logical_and(cnt < _NS, n < _N)

        def step(carry):
            n, cnt = carry
            ax = axb[pl.ds(n, 16)]
            ay = ayb[pl.ds(n, 16)]
            az = azb[pl.ds(n, 16)]
            p2 = p2b[pl.ds(n, 16)]
            dot = (ax * bcx + ay * bcy) + az * bcz
            sqr = (c2 + p2) - 2.0 * dot
            msk = jnp.logical_not(sqr > r2)
            mi = msk.astype(jnp.int32)
            pos = cnt + plsc.cumsum(mi) - mi
            plsc.store_scatter(idxb, [pos], lane16 + n, mask=msk)
            return (n + 16, cnt + jnp.sum(mi))

        _, cnt = lax.while_loop(cond, step, (jnp.int32(0), jnp.int32(0)))

        first = idxb[pl.ds(0, 16)][0]
        q0 = jnp.where(lane16 < cnt, idxb[pl.ds(0, 16)], first)
        q1 = jnp.where(lane16 + 16 < cnt, idxb[pl.ds(16, 16)], first)
        base = b * _N
        gidx[pl.ds(0, 16)] = q0 + base
        gidx[pl.ds(16, 16)] = q1 + base
        pltpu.async_copy(gh.at[gidx], gbuf, sem).wait()

        c0 = ccx * wxb[0, pl.ds(0, 16)] + ccy * wxb[1, pl.ds(0, 16)] + ccz * wxb[2, pl.ds(0, 16)]
        c1 = ccx * wxb[0, pl.ds(16, 16)] + ccy * wxb[1, pl.ds(16, 16)] + ccz * wxb[2, pl.ds(16, 16)]
        c2 = ccx * wxb[0, pl.ds(32, 16)] + ccy * wxb[1, pl.ds(32, 16)] + ccz * wxb[2, pl.ds(32, 16)]
        c3 = ccx * wxb[0, pl.ds(48, 16)] + ccy * wxb[1, pl.ds(48, 16)] + ccz * wxb[2, pl.ds(48, 16)]

        zero = jnp.zeros((16,), jnp.float32)
        ninf = jnp.full((16,), -jnp.inf, jnp.float32)

        def rowstep(j, acc):
            a0, a1, x2, x3 = acc
            h0 = gbuf[j, pl.ds(0, 16)] - c0
            h1 = gbuf[j, pl.ds(16, 16)] - c1
            h2 = gbuf[j, pl.ds(32, 16)] - c2
            h3 = gbuf[j, pl.ds(48, 16)] - c3
            h0 = jnp.where(h0 >= 0.0, h0, 0.01 * h0)
            h1 = jnp.where(h1 >= 0.0, h1, 0.01 * h1)
            h2 = jnp.where(h2 >= 0.0, h2, 0.01 * h2)
            h3 = jnp.where(h3 >= 0.0, h3, 0.01 * h3)
            return (a0 + h0, a1 + h1, jnp.maximum(x2, h2), jnp.maximum(x3, h3))

        a0, a1, x2, x3 = lax.fori_loop(
            0, _NS, rowstep, (zero, zero, ninf, ninf), unroll=4)
        outb[i, pl.ds(0, 16)] = a0 * inv_ns
        outb[i, pl.ds(16, 16)] = a1 * inv_ns
        outb[i, pl.ds(32, 16)] = x2
        outb[i, pl.ds(48, 16)] = x3
        return 0

    lax.fori_loop(0, _PPB, group, 0)

    pltpu.sync_copy(outb, outh.at[b, pl.ds(m0, _PPB)])
    pltpu.sync_copy(nxb.at[pl.ds(0, _PPB)], nxh.at[b, pl.ds(m0, _PPB)])
    pltpu.sync_copy(nyb.at[pl.ds(0, _PPB)], nyh.at[b, pl.ds(m0, _PPB)])
    pltpu.sync_copy(nzb.at[pl.ds(0, _PPB)], nzh.at[b, pl.ds(m0, _PPB)])
    pltpu.sync_copy(nm0b, nm0h.at[b, pl.ds(m0, _PPB)])
    pltpu.sync_copy(nm1b, nm1h.at[b, pl.ds(m0, _PPB)])
    pltpu.sync_copy(nkmb, nkmh.at[b, pl.ds(m0, _PPB)])


def _sc_group(xyzT, g2, fpsidx, mask0, mask1, km, wxt):
    f32, i32 = jnp.float32, jnp.int32
    out_type = [
        jax.ShapeDtypeStruct((_B, _M, _OUT), f32),
        jax.ShapeDtypeStruct((_B, _M), f32),
        jax.ShapeDtypeStruct((_B, _M), f32),
        jax.ShapeDtypeStruct((_B, _M), f32),
        jax.ShapeDtypeStruct((_B, _M), i32),
        jax.ShapeDtypeStruct((_B, _M), i32),
        jax.ShapeDtypeStruct((_B, _M), i32),
    ]
    scratch = [
        pltpu.VMEM((_N,), f32),
        pltpu.VMEM((_N,), f32),
        pltpu.VMEM((_N,), f32),
        pltpu.VMEM((_N,), f32),
        pltpu.VMEM((_N,), f32),
        pltpu.VMEM((_N,), f32),
        pltpu.VMEM((_N,), f32),
        pltpu.VMEM((_N,), i32),
        pltpu.VMEM((_N,), i32),
        pltpu.VMEM((_N,), i32),
        pltpu.VMEM((_PPB,), i32),
        pltpu.VMEM((3, _OUT), f32),
        pltpu.VMEM((_PPB + 16,), f32),
        pltpu.VMEM((_PPB + 16,), f32),
        pltpu.VMEM((_PPB + 16,), f32),
        pltpu.VMEM((_PPB,), i32),
        pltpu.VMEM((_PPB,), i32),
        pltpu.VMEM((_PPB,), i32),
        pltpu.VMEM((64,), i32),
        pltpu.VMEM((_NS,), i32),
        pltpu.VMEM((_NS, _OUT), f32),
        pltpu.VMEM((_PPB, _OUT), f32),
        pltpu.SemaphoreType.DMA,
    ]
    mesh = plsc.VectorSubcoreMesh(core_axis_name="c", subcore_axis_name="s")
    fn = pl.kernel(_sc_body, out_type=out_type, mesh=mesh,
                   scratch_types=scratch,
                   compiler_params=pltpu.CompilerParams(
                       needs_layout_passes=False,
                       use_tc_tiling_on_sc=False))
    return fn(xyzT, g2, fpsidx, mask0, mask1, km, wxt)


@jax.jit
def kernel(feat, xyz, masks, key_mask, W, b):
    xyzT = jnp.transpose(xyz, (2, 0, 1))
    kmf = key_mask.astype(jnp.float32)
    idx = _fps(xyzT, kmf)

    wft = jnp.transpose(W[:, :_IN])
    wxt = jnp.transpose(W[:, _IN:])
    g = _xform(feat, xyz, wft, wxt, b.reshape(1, _OUT))
    g2 = g.reshape(_B * _N, _OUT)

    out, nx, ny, nz, nm0, nm1, nkm = _sc_group(
        xyzT, g2, idx, masks[0], masks[1], key_mask, wxt)
    new_xyz = jnp.stack([nx, ny, nz], axis=-1)
    new_masks = jnp.stack([nm0, nm1], axis=0)
    return (out, new_xyz, new_masks, nkm)


# speedup vs baseline: 20.9096x; 1.3055x over previous
import functools

import jax
import jax.numpy as jnp
from jax import lax
from jax.experimental import pallas as pl
from jax.experimental.pallas import tpu as pltpu
from jax.experimental.pallas import tpu_sc as plsc

_B, _N, _IN, _OUT = 4, 8192, 64, 64
_M, _RADIUS, _NS = 2048, 0.2, 32
_R2 = _RADIUS * _RADIUS
_NW = 32
_GPW = (_B * _M) // _NW
_PPB = _M // (_NW // _B)


def _fps_body(xyzT_ref, kmf_ref, idx_ref, sx_ref, sy_ref, sz_ref, dist_ref):
    lanei = lax.broadcasted_iota(jnp.int32, (_B, _N), 1)
    lanef = lanei.astype(jnp.float32)
    off = kmf_ref[...] * ((lanef + 10.0) * 10.0)
    sx_ref[...] = xyzT_ref[0] + off
    sy_ref[...] = xyzT_ref[1] + off
    sz_ref[...] = xyzT_ref[2] + off
    dist_ref[...] = jnp.full((_B, _N), 1e10, jnp.float32)

    lanem = lax.broadcasted_iota(jnp.int32, (_B, _M), 1)

    def body(i, far):
        idx_ref[...] = jnp.where(lanem == i, far, idx_ref[...])
        onehot = lanei == far
        sx = sx_ref[...]
        sy = sy_ref[...]
        sz = sz_ref[...]
        cx = jnp.sum(jnp.where(onehot, sx, 0.0), axis=1, keepdims=True)
        cy = jnp.sum(jnp.where(onehot, sy, 0.0), axis=1, keepdims=True)
        cz = jnp.sum(jnp.where(onehot, sz, 0.0), axis=1, keepdims=True)
        dx = sx - cx
        dy = sy - cy
        dz = sz - cz
        d = (dx * dx + dy * dy) + dz * dz
        dist = jnp.minimum(dist_ref[...], d)
        dist_ref[...] = dist
        maxv = jnp.max(dist, axis=1, keepdims=True)
        return jnp.min(jnp.where(dist == maxv, lanei, _N), axis=1, keepdims=True)

    lax.fori_loop(0, _M, body, jnp.zeros((_B, 1), jnp.int32))


def _fps(xyzT, kmf):
    return pl.pallas_call(
        _fps_body,
        out_shape=jax.ShapeDtypeStruct((_B, _M), jnp.int32),
        scratch_shapes=[pltpu.VMEM((_B, _N), jnp.float32)] * 4,
    )(xyzT, kmf)


def _xf_body(feat_ref, xyz_ref, wft_ref, wxt_ref, bias_ref, g_ref):
    g = jnp.dot(feat_ref[0], wft_ref[...], preferred_element_type=jnp.float32)
    x3 = xyz_ref[0]
    g = g + x3[:, 0:1] * wxt_ref[0:1, :]
    g = g + x3[:, 1:2] * wxt_ref[1:2, :]
    g = g + x3[:, 2:3] * wxt_ref[2:3, :]
    g_ref[0] = g + bias_ref[...]


def _xform(feat, xyz, wft, wxt, bias):
    return pl.pallas_call(
        _xf_body,
        grid=(_B,),
        in_specs=[
            pl.BlockSpec((1, _N, _IN), lambda b: (b, 0, 0)),
            pl.BlockSpec((1, _N, 3), lambda b: (b, 0, 0)),
            pl.BlockSpec((_IN, _OUT), lambda b: (0, 0)),
            pl.BlockSpec((3, _OUT), lambda b: (0, 0)),
            pl.BlockSpec((1, _OUT), lambda b: (0, 0)),
        ],
        out_specs=pl.BlockSpec((1, _N, _OUT), lambda b: (b, 0, 0)),
        out_shape=jax.ShapeDtypeStruct((_B, _N, _OUT), jnp.float32),
    )(feat, xyz, wft, wxt, bias)


def _bf16_round(v):
    u = plsc.bitcast(v, jnp.uint32)
    u = u + jnp.uint32(0x7FFF) + ((u >> jnp.uint32(16)) & jnp.uint32(1))
    u = u & jnp.uint32(0xFFFF0000)
    return plsc.bitcast(u, jnp.float32)


def _sc_body(xyzT, gh, fih, m0h, m1h, kmh, wxth,
             outh, nxh, nyh, nzh, nm0h, nm1h, nkmh,
             xb, yb, zb, axb, ayb, azb, p2b, mb0, mb1, kmb, fpsb, wxb,
             nxb, nyb, nzb, nm0b, nm1b, nkmb,
             idxb, gidx, gbuf, outb, sem0, sem1):
    cid = lax.axis_index("c")
    sid = lax.axis_index("s")
    w = sid * 2 + cid
    b = w // (_NW // _B)
    m0 = (w % (_NW // _B)) * _PPB

    pltpu.sync_copy(xyzT.at[0, b], xb)
    pltpu.sync_copy(xyzT.at[1, b], yb)
    pltpu.sync_copy(xyzT.at[2, b], zb)
    pltpu.sync_copy(m0h.at[b], mb0)
    pltpu.sync_copy(m1h.at[b], mb1)
    pltpu.sync_copy(kmh.at[b], kmb)
    pltpu.sync_copy(fih.at[b, pl.ds(m0, _PPB)], fpsb)
    pltpu.sync_copy(wxth, wxb)

    def prep(t, _):
        s = t * 16
        vx = xb[pl.ds(s, 16)]
        vy = yb[pl.ds(s, 16)]
        vz = zb[pl.ds(s, 16)]
        axb[pl.ds(s, 16)] = _bf16_round(vx)
        ayb[pl.ds(s, 16)] = _bf16_round(vy)
        azb[pl.ds(s, 16)] = _bf16_round(vz)
        p2b[pl.ds(s, 16)] = (vx * vx + vy * vy) + vz * vz
        return 0

    lax.fori_loop(0, _N // 16, prep, 0)

    def mg(t, _):
        s = t * 16
        iv = fpsb[pl.ds(s, 16)]
        nxb[pl.ds(s, 16)] = plsc.load_gather(xb, [iv])
        nyb[pl.ds(s, 16)] = plsc.load_gather(yb, [iv])
        nzb[pl.ds(s, 16)] = plsc.load_gather(zb, [iv])
        nm0b[pl.ds(s, 16)] = plsc.load_gather(mb0, [iv])
        nm1b[pl.ds(s, 16)] = plsc.load_gather(mb1, [iv])
        nkmb[pl.ds(s, 16)] = plsc.load_gather(kmb, [iv])
        return 0

    lax.fori_loop(0, _PPB // 16, mg, 0)

    lane16 = lax.iota(jnp.int32, 16)
    r2 = jnp.float32(_R2)
    inv_ns = jnp.float32(1.0 / _NS)
    base = b * _N
    zero = jnp.zeros((16,), jnp.float32)
    ninf = jnp.full((16,), -jnp.inf, jnp.float32)

    def ballquery(i, slot):
        nvx = nxb[pl.ds(i, 16)]
        nvy = nyb[pl.ds(i, 16)]
        nvz = nzb[pl.ds(i, 16)]
        ccx = nvx[0]
        ccy = nvy[0]
        ccz = nvz[0]
        bcx = _bf16_round(nvx)[0]
        bcy = _bf16_round(nvy)[0]
        bcz = _bf16_round(nvz)[0]
        cn2 = (ccx * ccx + ccy * ccy) + ccz * ccz

        def cond(carry):
            n, cnt = carry
            return jnp.logical_and(cnt < _NS, n < _N)

        def step(carry):
            n, cnt = carry
            ax0 = axb[pl.ds(n, 16)]
            ay0 = ayb[pl.ds(n, 16)]
            az0 = azb[pl.ds(n, 16)]
            p20 = p2b[pl.ds(n, 16)]
            ax1 = axb[pl.ds(n + 16, 16)]
            ay1 = ayb[pl.ds(n + 16, 16)]
            az1 = azb[pl.ds(n + 16, 16)]
            p21 = p2b[pl.ds(n + 16, 16)]
            sqr0 = (cn2 + p20) - 2.0 * ((ax0 * bcx + ay0 * bcy) + az0 * bcz)
            sqr1 = (cn2 + p21) - 2.0 * ((ax1 * bcx + ay1 * bcy) + az1 * bcz)
            mk0 = jnp.logical_not(sqr0 > r2)
            mk1 = jnp.logical_not(sqr1 > r2)
            mi0 = mk0.astype(jnp.int32)
            mi1 = mk1.astype(jnp.int32)
            cv0 = plsc.cumsum(mi0)
            cv1 = plsc.cumsum(mi1)
            s0 = cv0[15]
            plsc.store_scatter(idxb, [cnt + cv0 - mi0], lane16 + n, mask=mk0)
            plsc.store_scatter(idxb, [(cnt + s0) + cv1 - mi1],
                               lane16 + (n + 16), mask=mk1)
            return (n + 32, (cnt + s0) + cv1[15])

        _, cnt = lax.while_loop(cond, step, (jnp.int32(0), jnp.int32(0)))

        first = idxb[pl.ds(0, 16)][0]
        q0 = jnp.where(lane16 < cnt, idxb[pl.ds(0, 16)], first)
        q1 = jnp.where(lane16 + 16 < cnt, idxb[pl.ds(16, 16)], first)
        gidx[slot, pl.ds(0, 16)] = q0 + base
        gidx[slot, pl.ds(16, 16)] = q1 + base
        c0 = ccx * wxb[0, pl.ds(0, 16)] + ccy * wxb[1, pl.ds(0, 16)] + ccz * wxb[2, pl.ds(0, 16)]
        c1 = ccx * wxb[0, pl.ds(16, 16)] + ccy * wxb[1, pl.ds(16, 16)] + ccz * wxb[2, pl.ds(16, 16)]
        c2 = ccx * wxb[0, pl.ds(32, 16)] + ccy * wxb[1, pl.ds(32, 16)] + ccz * wxb[2, pl.ds(32, 16)]
        c3 = ccx * wxb[0, pl.ds(48, 16)] + ccy * wxb[1, pl.ds(48, 16)] + ccz * wxb[2, pl.ds(48, 16)]
        return (c0, c1, c2, c3)

    def pool(slot, i, cx4):
        c0, c1, c2, c3 = cx4

        def rowstep(j, acc):
            a0, a1, x2, x3 = acc
            h0 = gbuf[slot, j, pl.ds(0, 16)] - c0
            h1 = gbuf[slot, j, pl.ds(16, 16)] - c1
            h2 = gbuf[slot, j, pl.ds(32, 16)] - c2
            h3 = gbuf[slot, j, pl.ds(48, 16)] - c3
            h0 = jnp.where(h0 >= 0.0, h0, 0.01 * h0)
            h1 = jnp.where(h1 >= 0.0, h1, 0.01 * h1)
            h2 = jnp.where(h2 >= 0.0, h2, 0.01 * h2)
            h3 = jnp.where(h3 >= 0.0, h3, 0.01 * h3)
            return (a0 + h0, a1 + h1, jnp.maximum(x2, h2), jnp.maximum(x3, h3))

        a0, a1, x2, x3 = lax.fori_loop(
            0, _NS, rowstep, (zero, zero, ninf, ninf), unroll=4)
        outb[i, pl.ds(0, 16)] = a0 * inv_ns
        outb[i, pl.ds(16, 16)] = a1 * inv_ns
        outb[i, pl.ds(32, 16)] = x2
        outb[i, pl.ds(48, 16)] = x3

    def pair(t, carry):
        i0 = 2 * t
        e = ballquery(i0, 0)
        pltpu.async_copy(gh.at[gidx.at[0]], gbuf.at[0], sem0)

        @pl.when(t > 0)
        def _():
            pltpu.make_async_copy(gh.at[gidx.at[1]], gbuf.at[1], sem1).wait()
            pool(1, i0 - 1, carry)

        f = ballquery(i0 + 1, 1)
        pltpu.async_copy(gh.at[gidx.at[1]], gbuf.at[1], sem1)
        pltpu.make_async_copy(gh.at[gidx.at[0]], gbuf.at[0], sem0).wait()
        pool(0, i0, e)
        return f

    fin = lax.fori_loop(0, _PPB // 2, pair, (zero, zero, zero, zero))
    pltpu.make_async_copy(gh.at[gidx.at[1]], gbuf.at[1], sem1).wait()
    pool(1, _PPB - 1, fin)

    pltpu.sync_copy(outb, outh.at[b, pl.ds(m0, _PPB)])
    pltpu.sync_copy(nxb.at[pl.ds(0, _PPB)], nxh.at[b, pl.ds(m0, _PPB)])
    pltpu.sync_copy(nyb.at[pl.ds(0, _PPB)], nyh.at[b, pl.ds(m0, _PPB)])
    pltpu.sync_copy(nzb.at[pl.ds(0, _PPB)], nzh.at[b, pl.ds(m0, _PPB)])
    pltpu.sync_copy(nm0b, nm0h.at[b, pl.ds(m0, _PPB)])
    pltpu.sync_copy(nm1b, nm1h.at[b, pl.ds(m0, _PPB)])
    pltpu.sync_copy(nkmb, nkmh.at[b, pl.ds(m0, _PPB)])


def _sc_group(xyzT, g2, fpsidx, mask0, mask1, km, wxt):
    f32, i32 = jnp.float32, jnp.int32
    out_type = [
        jax.ShapeDtypeStruct((_B, _M, _OUT), f32),
        jax.ShapeDtypeStruct((_B, _M), f32),
        jax.ShapeDtypeStruct((_B, _M), f32),
        jax.ShapeDtypeStruct((_B, _M), f32),
        jax.ShapeDtypeStruct((_B, _M), i32),
        jax.ShapeDtypeStruct((_B, _M), i32),
        jax.ShapeDtypeStruct((_B, _M), i32),
    ]
    scratch = [
        pltpu.VMEM((_N,), f32),
        pltpu.VMEM((_N,), f32),
        pltpu.VMEM((_N,), f32),
        pltpu.VMEM((_N,), f32),
        pltpu.VMEM((_N,), f32),
        pltpu.VMEM((_N,), f32),
        pltpu.VMEM((_N,), f32),
        pltpu.VMEM((_N,), i32),
        pltpu.VMEM((_N,), i32),
        pltpu.VMEM((_N,), i32),
        pltpu.VMEM((_PPB,), i32),
        pltpu.VMEM((3, _OUT), f32),
        pltpu.VMEM((_PPB + 16,), f32),
        pltpu.VMEM((_PPB + 16,), f32),
        pltpu.VMEM((_PPB + 16,), f32),
        pltpu.VMEM((_PPB,), i32),
        pltpu.VMEM((_PPB,), i32),
        pltpu.VMEM((_PPB,), i32),
        pltpu.VMEM((64,), i32),
        pltpu.VMEM((2, _NS), i32),
        pltpu.VMEM((2, _NS, _OUT), f32),
        pltpu.VMEM((_PPB, _OUT), f32),
        pltpu.SemaphoreType.DMA,
        pltpu.SemaphoreType.DMA,
    ]
    mesh = plsc.VectorSubcoreMesh(core_axis_name="c", subcore_axis_name="s")
    fn = pl.kernel(_sc_body, out_type=out_type, mesh=mesh,
                   scratch_types=scratch,
                   compiler_params=pltpu.CompilerParams(
                       needs_layout_passes=False,
                       use_tc_tiling_on_sc=False))
    return fn(xyzT, g2, fpsidx, mask0, mask1, km, wxt)


@jax.jit
def kernel(feat, xyz, masks, key_mask, W, b):
    xyzT = jnp.transpose(xyz, (2, 0, 1))
    kmf = key_mask.astype(jnp.float32)
    idx = _fps(xyzT, kmf)

    wft = jnp.transpose(W[:, :_IN])
    wxt = jnp.transpose(W[:, _IN:])
    g = _xform(feat, xyz, wft, wxt, b.reshape(1, _OUT))
    g2 = g.reshape(_B * _N, _OUT)

    out, nx, ny, nz, nm0, nm1, nkm = _sc_group(
        xyzT, g2, idx, masks[0], masks[1], key_mask, wxt)
    new_xyz = jnp.stack([nx, ny, nz], axis=-1)
    new_masks = jnp.stack([nm0, nm1], axis=0)
    return (out, new_xyz, new_masks, nkm)


# speedup vs baseline: 24.5844x; 1.1757x over previous
import functools

import jax
import jax.numpy as jnp
from jax import lax
from jax.experimental import pallas as pl
from jax.experimental.pallas import tpu as pltpu
from jax.experimental.pallas import tpu_sc as plsc

_B, _N, _IN, _OUT = 4, 8192, 64, 64
_M, _RADIUS, _NS = 2048, 0.2, 32
_R2 = _RADIUS * _RADIUS
_NW = 32
_GPW = (_B * _M) // _NW
_PPB = _M // (_NW // _B)


_SL = _N // 128


def _fps_body(xyzT_ref, kmf_ref, idx_ref, sx_ref, sy_ref, sz_ref, dist_ref):
    shp = (_B, _SL, 128)
    lanei = (lax.broadcasted_iota(jnp.int32, shp, 1) * 128
             + lax.broadcasted_iota(jnp.int32, shp, 2))
    lanef = lanei.astype(jnp.float32)
    off = kmf_ref[...] * ((lanef + 10.0) * 10.0)
    sx_ref[...] = xyzT_ref[0] + off
    sy_ref[...] = xyzT_ref[1] + off
    sz_ref[...] = xyzT_ref[2] + off
    dist_ref[...] = jnp.full(shp, 1e10, jnp.float32)

    lane128 = lax.broadcasted_iota(jnp.int32, (_B, 1, 128), 2)

    def inner(j, carry):
        far, colbuf = carry
        colbuf = jnp.where(lane128 == j, far, colbuf)
        onehot = lanei == far
        sx = sx_ref[...]
        sy = sy_ref[...]
        sz = sz_ref[...]
        cx = jnp.sum(jnp.where(onehot, sx, 0.0), axis=(1, 2), keepdims=True)
        cy = jnp.sum(jnp.where(onehot, sy, 0.0), axis=(1, 2), keepdims=True)
        cz = jnp.sum(jnp.where(onehot, sz, 0.0), axis=(1, 2), keepdims=True)
        dx = sx - cx
        dy = sy - cy
        dz = sz - cz
        d = (dx * dx + dy * dy) + dz * dz
        dist = jnp.minimum(dist_ref[...], d)
        dist_ref[...] = dist
        maxv = jnp.max(dist, axis=(1, 2), keepdims=True)
        far = jnp.min(jnp.where(dist == maxv, lanei, _N), axis=(1, 2),
                      keepdims=True)
        return far, colbuf

    def outer(o, far):
        far, colbuf = lax.fori_loop(
            0, 128, inner, (far, jnp.zeros((_B, 1, 128), jnp.int32)))
        idx_ref[:, pl.ds(pl.multiple_of(o * 128, 128), 128)] = (
            colbuf.reshape(_B, 128))
        return far

    lax.fori_loop(0, _M // 128, outer, jnp.zeros((_B, 1, 1), jnp.int32))


def _fps(xyzT4, kmf4):
    return pl.pallas_call(
        _fps_body,
        out_shape=jax.ShapeDtypeStruct((_B, _M), jnp.int32),
        scratch_shapes=[pltpu.VMEM((_B, _SL, 128), jnp.float32)] * 4,
    )(xyzT4, kmf4)


def _xf_body(feat_ref, xyz_ref, wft_ref, wxt_ref, bias_ref, g_ref):
    g = jnp.dot(feat_ref[0], wft_ref[...], preferred_element_type=jnp.float32)
    x3 = xyz_ref[0]
    g = g + x3[:, 0:1] * wxt_ref[0:1, :]
    g = g + x3[:, 1:2] * wxt_ref[1:2, :]
    g = g + x3[:, 2:3] * wxt_ref[2:3, :]
    g_ref[0] = g + bias_ref[...]


def _xform(feat, xyz, wft, wxt, bias):
    return pl.pallas_call(
        _xf_body,
        grid=(_B,),
        in_specs=[
            pl.BlockSpec((1, _N, _IN), lambda b: (b, 0, 0)),
            pl.BlockSpec((1, _N, 3), lambda b: (b, 0, 0)),
            pl.BlockSpec((_IN, _OUT), lambda b: (0, 0)),
            pl.BlockSpec((3, _OUT), lambda b: (0, 0)),
            pl.BlockSpec((1, _OUT), lambda b: (0, 0)),
        ],
        out_specs=pl.BlockSpec((1, _N, _OUT), lambda b: (b, 0, 0)),
        out_shape=jax.ShapeDtypeStruct((_B, _N, _OUT), jnp.float32),
    )(feat, xyz, wft, wxt, bias)


def _bf16_round(v):
    u = plsc.bitcast(v, jnp.uint32)
    u = u + jnp.uint32(0x7FFF) + ((u >> jnp.uint32(16)) & jnp.uint32(1))
    u = u & jnp.uint32(0xFFFF0000)
    return plsc.bitcast(u, jnp.float32)


def _sc_body(xyzT, gh, fih, m0h, m1h, kmh, wxth,
             outh, nxh, nyh, nzh, nm0h, nm1h, nkmh,
             xb, yb, zb, axb, ayb, azb, p2b, mb0, mb1, kmb, fpsb, wxb,
             nxb, nyb, nzb, nm0b, nm1b, nkmb,
             idxb, gidx, gbuf, outb, sem0, sem1):
    cid = lax.axis_index("c")
    sid = lax.axis_index("s")
    w = sid * 2 + cid
    b = w // (_NW // _B)
    m0 = (w % (_NW // _B)) * _PPB

    pltpu.sync_copy(xyzT.at[0, b], xb)
    pltpu.sync_copy(xyzT.at[1, b], yb)
    pltpu.sync_copy(xyzT.at[2, b], zb)
    pltpu.sync_copy(m0h.at[b], mb0)
    pltpu.sync_copy(m1h.at[b], mb1)
    pltpu.sync_copy(kmh.at[b], kmb)
    pltpu.sync_copy(fih.at[b, pl.ds(m0, _PPB)], fpsb)
    pltpu.sync_copy(wxth, wxb)

    def prep(t, _):
        s = t * 16
        vx = xb[pl.ds(s, 16)]
        vy = yb[pl.ds(s, 16)]
        vz = zb[pl.ds(s, 16)]
        axb[pl.ds(s, 16)] = _bf16_round(vx)
        ayb[pl.ds(s, 16)] = _bf16_round(vy)
        azb[pl.ds(s, 16)] = _bf16_round(vz)
        p2b[pl.ds(s, 16)] = (vx * vx + vy * vy) + vz * vz
        return 0

    lax.fori_loop(0, _N // 16, prep, 0)

    def mg(t, _):
        s = t * 16
        iv = fpsb[pl.ds(s, 16)]
        nxb[pl.ds(s, 16)] = plsc.load_gather(xb, [iv])
        nyb[pl.ds(s, 16)] = plsc.load_gather(yb, [iv])
        nzb[pl.ds(s, 16)] = plsc.load_gather(zb, [iv])
        nm0b[pl.ds(s, 16)] = plsc.load_gather(mb0, [iv])
        nm1b[pl.ds(s, 16)] = plsc.load_gather(mb1, [iv])
        nkmb[pl.ds(s, 16)] = plsc.load_gather(kmb, [iv])
        return 0

    lax.fori_loop(0, _PPB // 16, mg, 0)

    lane16 = lax.iota(jnp.int32, 16)
    r2 = jnp.float32(_R2)
    inv_ns = jnp.float32(1.0 / _NS)
    base = b * _N
    zero = jnp.zeros((16,), jnp.float32)
    ninf = jnp.full((16,), -jnp.inf, jnp.float32)

    def ballquery(i, slot):
        nvx = nxb[pl.ds(i, 16)]
        nvy = nyb[pl.ds(i, 16)]
        nvz = nzb[pl.ds(i, 16)]
        ccx = nvx[0]
        ccy = nvy[0]
        ccz = nvz[0]
        bcx = _bf16_round(nvx)[0]
        bcy = _bf16_round(nvy)[0]
        bcz = _bf16_round(nvz)[0]
        cn2 = (ccx * ccx + ccy * ccy) + ccz * ccz

        def cond(carry):
            n, cnt = carry
            return jnp.logical_and(cnt < _NS, n < _N)

        def step(carry):
            n, cnt = carry
            ax0 = axb[pl.ds(n, 16)]
            ay0 = ayb[pl.ds(n, 16)]
            az0 = azb[pl.ds(n, 16)]
            p20 = p2b[pl.ds(n, 16)]
            ax1 = axb[pl.ds(n + 16, 16)]
            ay1 = ayb[pl.ds(n + 16, 16)]
            az1 = azb[pl.ds(n + 16, 16)]
            p21 = p2b[pl.ds(n + 16, 16)]
            sqr0 = (cn2 + p20) - 2.0 * ((ax0 * bcx + ay0 * bcy) + az0 * bcz)
            sqr1 = (cn2 + p21) - 2.0 * ((ax1 * bcx + ay1 * bcy) + az1 * bcz)
            mk0 = jnp.logical_not(sqr0 > r2)
            mk1 = jnp.logical_not(sqr1 > r2)
            mi0 = mk0.astype(jnp.int32)
            mi1 = mk1.astype(jnp.int32)
            cv0 = plsc.cumsum(mi0)
            cv1 = plsc.cumsum(mi1)
            s0 = cv0[15]
            plsc.store_scatter(idxb, [cnt + cv0 - mi0], lane16 + n, mask=mk0)
            plsc.store_scatter(idxb, [(cnt + s0) + cv1 - mi1],
                               lane16 + (n + 16), mask=mk1)
            return (n + 32, (cnt + s0) + cv1[15])

        _, cnt = lax.while_loop(cond, step, (jnp.int32(0), jnp.int32(0)))

        first = idxb[pl.ds(0, 16)][0]
        q0 = jnp.where(lane16 < cnt, idxb[pl.ds(0, 16)], first)
        q1 = jnp.where(lane16 + 16 < cnt, idxb[pl.ds(16, 16)], first)
        gidx[slot, pl.ds(0, 16)] = q0 + base
        gidx[slot, pl.ds(16, 16)] = q1 + base
        c0 = ccx * wxb[0, pl.ds(0, 16)] + ccy * wxb[1, pl.ds(0, 16)] + ccz * wxb[2, pl.ds(0, 16)]
        c1 = ccx * wxb[0, pl.ds(16, 16)] + ccy * wxb[1, pl.ds(16, 16)] + ccz * wxb[2, pl.ds(16, 16)]
        c2 = ccx * wxb[0, pl.ds(32, 16)] + ccy * wxb[1, pl.ds(32, 16)] + ccz * wxb[2, pl.ds(32, 16)]
        c3 = ccx * wxb[0, pl.ds(48, 16)] + ccy * wxb[1, pl.ds(48, 16)] + ccz * wxb[2, pl.ds(48, 16)]
        return (c0, c1, c2, c3)

    def pool(slot, i, cx4):
        c0, c1, c2, c3 = cx4

        def rowstep(j, acc):
            a0, a1, x2, x3 = acc
            h0 = gbuf[slot, j, pl.ds(0, 16)] - c0
            h1 = gbuf[slot, j, pl.ds(16, 16)] - c1
            h2 = gbuf[slot, j, pl.ds(32, 16)] - c2
            h3 = gbuf[slot, j, pl.ds(48, 16)] - c3
            h0 = jnp.where(h0 >= 0.0, h0, 0.01 * h0)
            h1 = jnp.where(h1 >= 0.0, h1, 0.01 * h1)
            h2 = jnp.where(h2 >= 0.0, h2, 0.01 * h2)
            h3 = jnp.where(h3 >= 0.0, h3, 0.01 * h3)
            return (a0 + h0, a1 + h1, jnp.maximum(x2, h2), jnp.maximum(x3, h3))

        a0, a1, x2, x3 = lax.fori_loop(
            0, _NS, rowstep, (zero, zero, ninf, ninf), unroll=4)
        outb[i, pl.ds(0, 16)] = a0 * inv_ns
        outb[i, pl.ds(16, 16)] = a1 * inv_ns
        outb[i, pl.ds(32, 16)] = x2
        outb[i, pl.ds(48, 16)] = x3

    def pair(t, carry):
        i0 = 2 * t
        e = ballquery(i0, 0)
        pltpu.async_copy(gh.at[gidx.at[0]], gbuf.at[0], sem0)

        @pl.when(t > 0)
        def _():
            pltpu.make_async_copy(gh.at[gidx.at[1]], gbuf.at[1], sem1).wait()
            pool(1, i0 - 1, carry)

        f = ballquery(i0 + 1, 1)
        pltpu.async_copy(gh.at[gidx.at[1]], gbuf.at[1], sem1)
        pltpu.make_async_copy(gh.at[gidx.at[0]], gbuf.at[0], sem0).wait()
        pool(0, i0, e)
        return f

    fin = lax.fori_loop(0, _PPB // 2, pair, (zero, zero, zero, zero))
    pltpu.make_async_copy(gh.at[gidx.at[1]], gbuf.at[1], sem1).wait()
    pool(1, _PPB - 1, fin)

    pltpu.sync_copy(outb, outh.at[b, pl.ds(m0, _PPB)])
    pltpu.sync_copy(nxb.at[pl.ds(0, _PPB)], nxh.at[b, pl.ds(m0, _PPB)])
    pltpu.sync_copy(nyb.at[pl.ds(0, _PPB)], nyh.at[b, pl.ds(m0, _PPB)])
    pltpu.sync_copy(nzb.at[pl.ds(0, _PPB)], nzh.at[b, pl.ds(m0, _PPB)])
    pltpu.sync_copy(nm0b, nm0h.at[b, pl.ds(m0, _PPB)])
    pltpu.sync_copy(nm1b, nm1h.at[b, pl.ds(m0, _PPB)])
    pltpu.sync_copy(nkmb, nkmh.at[b, pl.ds(m0, _PPB)])


def _sc_group(xyzT, g2, fpsidx, mask0, mask1, km, wxt):
    f32, i32 = jnp.float32, jnp.int32
    out_type = [
        jax.ShapeDtypeStruct((_B, _M, _OUT), f32),
        jax.ShapeDtypeStruct((_B, _M), f32),
        jax.ShapeDtypeStruct((_B, _M), f32),
        jax.ShapeDtypeStruct((_B, _M), f32),
        jax.ShapeDtypeStruct((_B, _M), i32),
        jax.ShapeDtypeStruct((_B, _M), i32),
        jax.ShapeDtypeStruct((_B, _M), i32),
    ]
    scratch = [
        pltpu.VMEM((_N,), f32),
        pltpu.VMEM((_N,), f32),
        pltpu.VMEM((_N,), f32),
        pltpu.VMEM((_N,), f32),
        pltpu.VMEM((_N,), f32),
        pltpu.VMEM((_N,), f32),
        pltpu.VMEM((_N,), f32),
        pltpu.VMEM((_N,), i32),
        pltpu.VMEM((_N,), i32),
        pltpu.VMEM((_N,), i32),
        pltpu.VMEM((_PPB,), i32),
        pltpu.VMEM((3, _OUT), f32),
        pltpu.VMEM((_PPB + 16,), f32),
        pltpu.VMEM((_PPB + 16,), f32),
        pltpu.VMEM((_PPB + 16,), f32),
        pltpu.VMEM((_PPB,), i32),
        pltpu.VMEM((_PPB,), i32),
        pltpu.VMEM((_PPB,), i32),
        pltpu.VMEM((64,), i32),
        pltpu.VMEM((2, _NS), i32),
        pltpu.VMEM((2, _NS, _OUT), f32),
        pltpu.VMEM((_PPB, _OUT), f32),
        pltpu.SemaphoreType.DMA,
        pltpu.SemaphoreType.DMA,
    ]
    mesh = plsc.VectorSubcoreMesh(core_axis_name="c", subcore_axis_name="s")
    fn = pl.kernel(_sc_body, out_type=out_type, mesh=mesh,
                   scratch_types=scratch,
                   compiler_params=pltpu.CompilerParams(
                       needs_layout_passes=False,
                       use_tc_tiling_on_sc=False))
    return fn(xyzT, g2, fpsidx, mask0, mask1, km, wxt)


@jax.jit
def kernel(feat, xyz, masks, key_mask, W, b):
    xyzT = jnp.transpose(xyz, (2, 0, 1))
    kmf = key_mask.astype(jnp.float32)
    idx = _fps(xyzT.reshape(3, _B, _SL, 128),
               kmf.reshape(_B, _SL, 128))

    wft = jnp.transpose(W[:, :_IN])
    wxt = jnp.transpose(W[:, _IN:])
    g = _xform(feat, xyz, wft, wxt, b.reshape(1, _OUT))
    g2 = g.reshape(_B * _N, _OUT)

    out, nx, ny, nz, nm0, nm1, nkm = _sc_group(
        xyzT, g2, idx, masks[0], masks[1], key_mask, wxt)
    new_xyz = jnp.stack([nx, ny, nz], axis=-1)
    new_masks = jnp.stack([nm0, nm1], axis=0)
    return (out, new_xyz, new_masks, nkm)
